# baseline (speedup 1.0000x reference)
import functools

import jax
import jax.numpy as jnp
from jax.experimental import pallas as pl
from jax.experimental.pallas import tpu as pltpu

TOP_K = 64
NEG_BIG = -1000000000.0
PACK = 4


def _select_kernel(x_ref, w_impT_ref, b_imp_ref, wk_ref, bk_ref, wv_ref,
                   bv_ref, idx_ref, kblk_ref, vblk_ref, xcopy_ref, s_ref,
                   *, k, heads, dh, stile, nst):
    t = pl.program_id(1)
    S = stile * nst
    D = x_ref.shape[1]
    xt = x_ref[...]
    xcopy_ref[pl.ds(t * stile, stile), :] = xt
    s_ref[pl.ds(t, 1), :] = jax.lax.dot_general(
        w_impT_ref[...], xt,
        dimension_numbers=(((1,), (1,)), ((), ())),
        preferred_element_type=jnp.float32) + b_imp_ref[...]

    @pl.when(t == nst - 1)
    def _():
        scores = s_ref[...]
        lin = (jax.lax.broadcasted_iota(jnp.int32, (nst, stile), 0) * stile +
               jax.lax.broadcasted_iota(jnp.int32, (nst, stile), 1))
        k_iota = jax.lax.broadcasted_iota(jnp.int32, (1, k), 1)
        kcol_iota = jax.lax.broadcasted_iota(jnp.int32, (k, 1), 0)

        def body(i, carry):
            sc, acc, acc_col = carry
            m = jnp.max(sc)
            cand = jnp.where(sc == m, lin, S)
            idx = jnp.min(cand).astype(jnp.int32)
            acc = jnp.where(k_iota == i, idx, acc)
            acc_col = jnp.where(kcol_iota == i, idx, acc_col)
            sc = jnp.where(lin == idx, -jnp.inf, sc)
            return sc, acc, acc_col

        acc0 = jnp.zeros((1, k), dtype=jnp.int32)
        acc_col0 = jnp.zeros((k, 1), dtype=jnp.int32)
        _, acc, acc_col = jax.lax.fori_loop(
            0, k, body, (scores, acc0, acc_col0))
        idx_ref[...] = acc
        sel_lane = jax.lax.broadcasted_iota(jnp.int32, (k, S), 1)
        onehot = (sel_lane == acc_col).astype(jnp.bfloat16)
        xb = xcopy_ref[...].astype(jnp.bfloat16)
        sel = jnp.dot(onehot, xb,
                      preferred_element_type=jnp.float32).astype(jnp.bfloat16)
        wk = wk_ref[...].astype(jnp.bfloat16)
        wv = wv_ref[...].astype(jnp.bfloat16)
        kp = (jnp.dot(sel, wk, preferred_element_type=jnp.float32)
              + bk_ref[...]).astype(jnp.bfloat16)
        vp = (jnp.dot(sel, wv, preferred_element_type=jnp.float32)
              + bv_ref[...]).astype(jnp.bfloat16)
        zero = jnp.zeros((k, dh), dtype=jnp.bfloat16)
        for g in range(heads // PACK):
            krows, vrows = [], []
            for hh in range(PACK):
                h = g * PACK + hh
                kb = [zero] * PACK
                vb = [zero] * PACK
                kb[hh] = kp[:, h * dh:(h + 1) * dh]
                vb[hh] = vp[:, h * dh:(h + 1) * dh]
                krows.append(jnp.concatenate(kb, axis=1))
                vrows.append(jnp.concatenate(vb, axis=1))
            kblk_ref[g] = jnp.concatenate(krows, axis=0)
            vblk_ref[g] = jnp.concatenate(vrows, axis=0)


def _attn_kernel(x_ref, wq_ref, bq_ref, kblk_ref, vblk_ref, idx_ref,
                 segdown_ref, segup_ref, wo_ref, bo_ref, out_ref,
                 *, heads, dh, tile, k):
    t = pl.program_id(1)
    xt = x_ref[...].astype(jnp.bfloat16)
    wq = wq_ref[...].astype(jnp.bfloat16)
    q = jnp.dot(xt, wq, preferred_element_type=jnp.float32) + bq_ref[...]
    q = q.astype(jnp.bfloat16)
    grp = PACK * dh
    ngrp = heads // PACK
    scale = 1.0 / jnp.sqrt(jnp.asarray(dh, dtype=jnp.float32))
    kpos = idx_ref[...]
    kpos_g = jnp.concatenate([kpos] * PACK, axis=1)
    qpos = t * tile + jax.lax.broadcasted_iota(jnp.int32, (tile, 1), 0)
    mask_g = qpos >= kpos_g
    sg = []
    for g in range(ngrp):
        s = jax.lax.dot_general(
            q[:, g * grp:(g + 1) * grp], kblk_ref[g],
            dimension_numbers=(((1,), (1,)), ((), ())),
            preferred_element_type=jnp.float32) * scale
        sg.append(jnp.where(mask_g, s, NEG_BIG))
    m = jnp.maximum(
        jnp.maximum(jnp.max(sg[0], axis=1, keepdims=True),
                    jnp.max(sg[1], axis=1, keepdims=True)),
        jnp.maximum(jnp.max(sg[2], axis=1, keepdims=True),
                    jnp.max(sg[3], axis=1, keepdims=True)))
    e = [jnp.exp(s - m) for s in sg]
    eb = [v.astype(jnp.bfloat16) for v in e]
    denom = jnp.dot(eb[0], segdown_ref[0], preferred_element_type=jnp.float32)
    for g in range(1, ngrp):
        denom += jnp.dot(eb[g], segdown_ref[g],
                         preferred_element_type=jnp.float32)
    recip = (1.0 / denom).astype(jnp.bfloat16)
    acc = None
    for g in range(ngrp):
        rexp = jnp.dot(recip, segup_ref[g],
                       preferred_element_type=jnp.float32)
        attn = (e[g] * rexp).astype(jnp.bfloat16)
        ctx = jnp.dot(attn, vblk_ref[g],
                      preferred_element_type=jnp.float32).astype(jnp.bfloat16)
        wo_g = wo_ref[g].astype(jnp.bfloat16)
        part = jnp.dot(ctx, wo_g, preferred_element_type=jnp.float32)
        acc = part if acc is None else acc + part
    out_ref[...] = acc + bo_ref[...]


def kernel(x, W_imp, b_imp, Wq, bq, Wk, bk, Wv, bv, Wo, bo):
    B, S, D = x.shape
    HD = Wq.shape[1]
    heads = 16
    dh = HD // heads
    k = min(TOP_K, S)
    tile = 1024
    nt = S // tile
    stile = 512
    nst = S // stile
    ngrp = heads // PACK
    grp = PACK * k

    w_impT = W_imp.T
    b_imp2 = b_imp.reshape(1, 1)
    bq2, bk2, bv2 = bq.reshape(1, HD), bk.reshape(1, HD), bv.reshape(1, HD)
    bo2 = bo.reshape(1, D)
    wo_r = Wo.reshape(ngrp, grp, D)
    lane_head = jnp.arange(grp) // k
    head_ids = jnp.arange(heads)
    segdown = jnp.stack([
        ((g * PACK + lane_head)[:, None] == head_ids[None, :])
        .astype(jnp.bfloat16) for g in range(ngrp)])
    segup = jnp.stack([
        (head_ids[:, None] == (g * PACK + lane_head)[None, :])
        .astype(jnp.bfloat16) for g in range(ngrp)])

    idx3, kblk, vblk = pl.pallas_call(
        functools.partial(_select_kernel, k=k, heads=heads, dh=dh,
                          stile=stile, nst=nst),
        grid=(B, nst),
        in_specs=[
            pl.BlockSpec((None, stile, D), lambda b, t: (b, t, 0)),
            pl.BlockSpec((1, D), lambda b, t: (0, 0)),
            pl.BlockSpec((1, 1), lambda b, t: (0, 0)),
            pl.BlockSpec((D, HD), lambda b, t: (0, 0)),
            pl.BlockSpec((1, HD), lambda b, t: (0, 0)),
            pl.BlockSpec((D, HD), lambda b, t: (0, 0)),
            pl.BlockSpec((1, HD), lambda b, t: (0, 0)),
        ],
        out_specs=[
            pl.BlockSpec((None, 1, k), lambda b, t: (b, 0, 0)),
            pl.BlockSpec((None, ngrp, grp, grp), lambda b, t: (b, 0, 0, 0)),
            pl.BlockSpec((None, ngrp, grp, grp), lambda b, t: (b, 0, 0, 0)),
        ],
        out_shape=[
            jax.ShapeDtypeStruct((B, 1, k), jnp.int32),
            jax.ShapeDtypeStruct((B, ngrp, grp, grp), jnp.bfloat16),
            jax.ShapeDtypeStruct((B, ngrp, grp, grp), jnp.bfloat16),
        ],
        scratch_shapes=[
            pltpu.VMEM((S, D), jnp.float32),
            pltpu.VMEM((nst, stile), jnp.float32),
        ],
    )(x, w_impT, b_imp2, Wk, bk2, Wv, bv2)

    out = pl.pallas_call(
        functools.partial(_attn_kernel, heads=heads, dh=dh, tile=tile, k=k),
        grid=(B, nt),
        in_specs=[
            pl.BlockSpec((None, tile, D), lambda b, t: (b, t, 0)),
            pl.BlockSpec((D, HD), lambda b, t: (0, 0)),
            pl.BlockSpec((1, HD), lambda b, t: (0, 0)),
            pl.BlockSpec((None, ngrp, grp, grp), lambda b, t: (b, 0, 0, 0)),
            pl.BlockSpec((None, ngrp, grp, grp), lambda b, t: (b, 0, 0, 0)),
            pl.BlockSpec((None, 1, k), lambda b, t: (b, 0, 0)),
            pl.BlockSpec((ngrp, grp, heads), lambda b, t: (0, 0, 0)),
            pl.BlockSpec((ngrp, heads, grp), lambda b, t: (0, 0, 0)),
            pl.BlockSpec((ngrp, grp, D), lambda b, t: (0, 0, 0)),
            pl.BlockSpec((1, D), lambda b, t: (0, 0)),
        ],
        out_specs=pl.BlockSpec((None, tile, D), lambda b, t: (b, t, 0)),
        out_shape=jax.ShapeDtypeStruct((B, S, D), jnp.float32),
    )(x, Wq, bq2, kblk, vblk, idx3, segdown, segup, wo_r, bo2)

    return out, idx3.reshape(B, k)


# speedup vs baseline: 1.0299x; 1.0299x over previous
import functools

import jax
import jax.numpy as jnp
from jax.experimental import pallas as pl
from jax.experimental.pallas import tpu as pltpu

TOP_K = 64
NEG_BIG = -1000000000.0
PACK = 4


def _select_kernel(x_ref, w_impT_ref, b_imp_ref, wk_ref, bk_ref, wv_ref,
                   bv_ref, wq_ref, bq_ref, idx_ref, kblk_ref, vblk_ref,
                   q_ref, xcopy_ref, s_ref, *, k, heads, dh, stile, nst):
    t = pl.program_id(1)
    S = stile * nst
    D = x_ref.shape[1]
    xt = x_ref[...]
    xcopy_ref[pl.ds(t * stile, stile), :] = xt
    s_ref[pl.ds(t, 1), :] = jax.lax.dot_general(
        w_impT_ref[...], xt,
        dimension_numbers=(((1,), (1,)), ((), ())),
        preferred_element_type=jnp.float32) + b_imp_ref[...]
    wq = wq_ref[...].astype(jnp.bfloat16)
    q = jnp.dot(xt.astype(jnp.bfloat16), wq,
                preferred_element_type=jnp.float32) + bq_ref[...]
    q_ref[...] = q.astype(jnp.bfloat16)

    @pl.when(t == nst - 1)
    def _():
        scores = s_ref[...]
        lin = (jax.lax.broadcasted_iota(jnp.int32, (nst, stile), 0) * stile +
               jax.lax.broadcasted_iota(jnp.int32, (nst, stile), 1))
        k_iota = jax.lax.broadcasted_iota(jnp.int32, (1, k), 1)
        kcol_iota = jax.lax.broadcasted_iota(jnp.int32, (k, 1), 0)

        def body(i, carry):
            sc, acc, acc_col = carry
            m = jnp.max(sc)
            cand = jnp.where(sc == m, lin, S)
            idx = jnp.min(cand).astype(jnp.int32)
            acc = jnp.where(k_iota == i, idx, acc)
            acc_col = jnp.where(kcol_iota == i, idx, acc_col)
            sc = jnp.where(lin == idx, -jnp.inf, sc)
            return sc, acc, acc_col

        acc0 = jnp.zeros((1, k), dtype=jnp.int32)
        acc_col0 = jnp.zeros((k, 1), dtype=jnp.int32)
        _, acc, acc_col = jax.lax.fori_loop(
            0, k, body, (scores, acc0, acc_col0))
        idx_ref[...] = acc
        sel_lane = jax.lax.broadcasted_iota(jnp.int32, (k, S), 1)
        onehot = (sel_lane == acc_col).astype(jnp.bfloat16)
        xb = xcopy_ref[...].astype(jnp.bfloat16)
        sel = jnp.dot(onehot, xb,
                      preferred_element_type=jnp.float32).astype(jnp.bfloat16)
        wk = wk_ref[...].astype(jnp.bfloat16)
        wv = wv_ref[...].astype(jnp.bfloat16)
        kp = (jnp.dot(sel, wk, preferred_element_type=jnp.float32)
              + bk_ref[...]).astype(jnp.bfloat16)
        vp = (jnp.dot(sel, wv, preferred_element_type=jnp.float32)
              + bv_ref[...]).astype(jnp.bfloat16)
        zero = jnp.zeros((k, dh), dtype=jnp.bfloat16)
        for g in range(heads // PACK):
            krows, vrows = [], []
            for hh in range(PACK):
                h = g * PACK + hh
                kb = [zero] * PACK
                vb = [zero] * PACK
                kb[hh] = kp[:, h * dh:(h + 1) * dh]
                vb[hh] = vp[:, h * dh:(h + 1) * dh]
                krows.append(jnp.concatenate(kb, axis=1))
                vrows.append(jnp.concatenate(vb, axis=1))
            kblk_ref[g] = jnp.concatenate(krows, axis=0)
            vblk_ref[g] = jnp.concatenate(vrows, axis=0)


def _attn_kernel(q_ref, kblk_ref, vblk_ref, idx_ref,
                 segdown_ref, segup_ref, wo_ref, bo_ref, out_ref,
                 *, heads, dh, tile, k):
    t = pl.program_id(1)
    q = q_ref[...]
    grp = PACK * dh
    ngrp = heads // PACK
    scale = 1.0 / jnp.sqrt(jnp.asarray(dh, dtype=jnp.float32))
    kpos = idx_ref[...]
    kpos_g = jnp.concatenate([kpos] * PACK, axis=1)
    qpos = t * tile + jax.lax.broadcasted_iota(jnp.int32, (tile, 1), 0)
    mask_g = qpos >= kpos_g
    sg = []
    for g in range(ngrp):
        s = jax.lax.dot_general(
            q[:, g * grp:(g + 1) * grp], kblk_ref[g],
            dimension_numbers=(((1,), (1,)), ((), ())),
            preferred_element_type=jnp.float32) * scale
        sg.append(jnp.where(mask_g, s, NEG_BIG))
    m = jnp.maximum(
        jnp.maximum(jnp.max(sg[0], axis=1, keepdims=True),
                    jnp.max(sg[1], axis=1, keepdims=True)),
        jnp.maximum(jnp.max(sg[2], axis=1, keepdims=True),
                    jnp.max(sg[3], axis=1, keepdims=True)))
    e = [jnp.exp(s - m) for s in sg]
    eb = [v.astype(jnp.bfloat16) for v in e]
    denom = jnp.dot(eb[0], segdown_ref[0], preferred_element_type=jnp.float32)
    for g in range(1, ngrp):
        denom += jnp.dot(eb[g], segdown_ref[g],
                         preferred_element_type=jnp.float32)
    recip = (1.0 / denom).astype(jnp.bfloat16)
    acc = None
    for g in range(ngrp):
        rexp = jnp.dot(recip, segup_ref[g],
                       preferred_element_type=jnp.float32)
        attn = (e[g] * rexp).astype(jnp.bfloat16)
        ctx = jnp.dot(attn, vblk_ref[g],
                      preferred_element_type=jnp.float32).astype(jnp.bfloat16)
        wo_g = wo_ref[g].astype(jnp.bfloat16)
        part = jnp.dot(ctx, wo_g, preferred_element_type=jnp.float32)
        acc = part if acc is None else acc + part
    out_ref[...] = acc + bo_ref[...]


def kernel(x, W_imp, b_imp, Wq, bq, Wk, bk, Wv, bv, Wo, bo):
    B, S, D = x.shape
    HD = Wq.shape[1]
    heads = 16
    dh = HD // heads
    k = min(TOP_K, S)
    tile = 1024
    nt = S // tile
    stile = 512
    nst = S // stile
    ngrp = heads // PACK
    grp = PACK * k

    w_impT = W_imp.T
    b_imp2 = b_imp.reshape(1, 1)
    bq2, bk2, bv2 = bq.reshape(1, HD), bk.reshape(1, HD), bv.reshape(1, HD)
    bo2 = bo.reshape(1, D)
    wo_r = Wo.reshape(ngrp, grp, D)
    lane_head = jnp.arange(grp) // k
    head_ids = jnp.arange(heads)
    segdown = jnp.stack([
        ((g * PACK + lane_head)[:, None] == head_ids[None, :])
        .astype(jnp.bfloat16) for g in range(ngrp)])
    segup = jnp.stack([
        (head_ids[:, None] == (g * PACK + lane_head)[None, :])
        .astype(jnp.bfloat16) for g in range(ngrp)])

    idx3, kblk, vblk, qall = pl.pallas_call(
        functools.partial(_select_kernel, k=k, heads=heads, dh=dh,
                          stile=stile, nst=nst),
        grid=(B, nst),
        in_specs=[
            pl.BlockSpec((None, stile, D), lambda b, t: (b, t, 0)),
            pl.BlockSpec((1, D), lambda b, t: (0, 0)),
            pl.BlockSpec((1, 1), lambda b, t: (0, 0)),
            pl.BlockSpec((D, HD), lambda b, t: (0, 0)),
            pl.BlockSpec((1, HD), lambda b, t: (0, 0)),
            pl.BlockSpec((D, HD), lambda b, t: (0, 0)),
            pl.BlockSpec((1, HD), lambda b, t: (0, 0)),
            pl.BlockSpec((D, HD), lambda b, t: (0, 0)),
            pl.BlockSpec((1, HD), lambda b, t: (0, 0)),
        ],
        out_specs=[
            pl.BlockSpec((None, 1, k), lambda b, t: (b, 0, 0)),
            pl.BlockSpec((None, ngrp, grp, grp), lambda b, t: (b, 0, 0, 0)),
            pl.BlockSpec((None, ngrp, grp, grp), lambda b, t: (b, 0, 0, 0)),
            pl.BlockSpec((None, stile, HD), lambda b, t: (b, t, 0)),
        ],
        out_shape=[
            jax.ShapeDtypeStruct((B, 1, k), jnp.int32),
            jax.ShapeDtypeStruct((B, ngrp, grp, grp), jnp.bfloat16),
            jax.ShapeDtypeStruct((B, ngrp, grp, grp), jnp.bfloat16),
            jax.ShapeDtypeStruct((B, S, HD), jnp.bfloat16),
        ],
        scratch_shapes=[
            pltpu.VMEM((S, D), jnp.float32),
            pltpu.VMEM((nst, stile), jnp.float32),
        ],
    )(x, w_impT, b_imp2, Wk, bk2, Wv, bv2, Wq, bq2)

    out = pl.pallas_call(
        functools.partial(_attn_kernel, heads=heads, dh=dh, tile=tile, k=k),
        grid=(B, nt),
        in_specs=[
            pl.BlockSpec((None, tile, HD), lambda b, t: (b, t, 0)),
            pl.BlockSpec((None, ngrp, grp, grp), lambda b, t: (b, 0, 0, 0)),
            pl.BlockSpec((None, ngrp, grp, grp), lambda b, t: (b, 0, 0, 0)),
            pl.BlockSpec((None, 1, k), lambda b, t: (b, 0, 0)),
            pl.BlockSpec((ngrp, grp, heads), lambda b, t: (0, 0, 0)),
            pl.BlockSpec((ngrp, heads, grp), lambda b, t: (0, 0, 0)),
            pl.BlockSpec((ngrp, grp, D), lambda b, t: (0, 0, 0)),
            pl.BlockSpec((1, D), lambda b, t: (0, 0)),
        ],
        out_specs=pl.BlockSpec((None, tile, D), lambda b, t: (b, t, 0)),
        out_shape=jax.ShapeDtypeStruct((B, S, D), jnp.float32),
    )(qall, kblk, vblk, idx3, segdown, segup, wo_r, bo2)

    return out, idx3.reshape(B, k)
